# Initial kernel scaffold; baseline (speedup 1.0000x reference)
import jax
import jax.numpy as jnp
from jax.experimental import pallas as pl


def kernel(x, edge_index, edge_label_index, W1, b1, W2, b2, W3, b3):
    raise NotImplementedError("write your pallas kernel here")


# speedup vs baseline: 5.5117x; 5.5117x over previous
import functools

import jax
import jax.numpy as jnp
from jax import lax
from jax.experimental import pallas as pl
from jax.experimental.pallas import tpu as pltpu
from jax.experimental.pallas import tpu_sc as plsc

N = 10000
NP = 10240
D = 128
NC = 2
NS = 16
NW = NC * NS
RPT = NP // NS

E = 320000
EB = 80
EP = NW * EB * 128
DUMMY = NP - 1

LBL = 100000
LB = 25
LP = NW * LB * 128

ROW_BLK = 1024


def _mesh():
    return plsc.VectorSubcoreMesh(
        core_axis_name="c", subcore_axis_name="s", num_cores=NC, num_subcores=NS
    )


def _zero_rows_buf(buf):
    def body(i, _):
        r = i // 8
        k = (i % 8) * 16
        buf[r, pl.ds(k, 16)] = jnp.zeros((16,), jnp.float32)
        return _
    lax.fori_loop(0, 1024, body, None)


def _sc_degree(dst_r):
    @functools.partial(
        pl.kernel,
        mesh=_mesh(),
        out_type=jax.ShapeDtypeStruct((NC, NP, D), jnp.float32),
        scratch_types=[
            pltpu.VMEM((EB, 128), jnp.int32),
            pltpu.VMEM((128, D), jnp.float32),
            pltpu.VMEM_SHARED((NP, D), jnp.float32),
        ],
    )
    def k(dst_hbm, out_hbm, idx_v, ones_v, acc):
        c = lax.axis_index("c")
        s = lax.axis_index("s")
        w = c * NS + s

        _zero_rows_buf(ones_v)
        for kk in range(RPT // 128):
            pltpu.sync_copy(ones_v, acc.at[pl.ds(s * RPT + kk * 128, 128)])

        def fill(i, _):
            r = i // 8
            kofs = (i % 8) * 16
            ones_v[r, pl.ds(kofs, 16)] = jnp.ones((16,), jnp.float32)
            return _
        lax.fori_loop(0, 1024, fill, None)
        pltpu.sync_copy(dst_hbm.at[w], idx_v)
        plsc.subcore_barrier()

        def body(j, _):
            pltpu.sync_copy(ones_v, acc.at[idx_v.at[j]], add=True)
            return _
        lax.fori_loop(0, EB, body, None)

        plsc.subcore_barrier()
        pltpu.sync_copy(acc.at[pl.ds(s * RPT, RPT)],
                        out_hbm.at[c, pl.ds(s * RPT, RPT)])

    return k(dst_r)


def _sc_propagate(src_r, dst_r, t):
    @functools.partial(
        pl.kernel,
        mesh=_mesh(),
        out_type=jax.ShapeDtypeStruct((NC, NP, D), jnp.float32),
        scratch_types=[
            pltpu.VMEM((EB, 128), jnp.int32),
            pltpu.VMEM((EB, 128), jnp.int32),
            pltpu.VMEM((128, D), jnp.float32),
            pltpu.VMEM_SHARED((NP, D), jnp.float32),
            pltpu.SemaphoreType.DMA,
        ],
    )
    def k(src_hbm, dst_hbm, t_hbm, out_hbm, idx_s, idx_d, rows, acc, sem):
        c = lax.axis_index("c")
        s = lax.axis_index("s")
        w = c * NS + s

        _zero_rows_buf(rows)
        for kk in range(RPT // 128):
            pltpu.sync_copy(rows, acc.at[pl.ds(s * RPT + kk * 128, 128)])
        pltpu.sync_copy(src_hbm.at[w], idx_s)
        pltpu.sync_copy(dst_hbm.at[w], idx_d)
        plsc.subcore_barrier()

        def body(j, _):
            pltpu.async_copy(t_hbm.at[idx_s.at[j]], rows, sem).wait()
            pltpu.sync_copy(rows, acc.at[idx_d.at[j]], add=True)
            return _
        lax.fori_loop(0, EB, body, None)

        plsc.subcore_barrier()
        pltpu.sync_copy(acc.at[pl.ds(s * RPT, RPT)],
                        out_hbm.at[c, pl.ds(s * RPT, RPT)])

    return k(src_r, dst_r, t)


def _sc_gather_prod(la_r, lb_r, h3):
    @functools.partial(
        pl.kernel,
        mesh=_mesh(),
        out_type=jax.ShapeDtypeStruct((LP, D), jnp.float32),
        scratch_types=[
            pltpu.VMEM((LB, 128), jnp.int32),
            pltpu.VMEM((LB, 128), jnp.int32),
            pltpu.VMEM((128, D), jnp.float32),
            pltpu.VMEM((128, D), jnp.float32),
            pltpu.SemaphoreType.DMA,
        ],
    )
    def k(la_hbm, lb_hbm, h3_hbm, out_hbm, ia, ib, abuf, bbuf, sem):
        c = lax.axis_index("c")
        s = lax.axis_index("s")
        w = c * NS + s
        base = w * LB * 128

        pltpu.sync_copy(la_hbm.at[w], ia)
        pltpu.sync_copy(lb_hbm.at[w], ib)

        def body(j, _):
            ca = pltpu.async_copy(h3_hbm.at[ia.at[j]], abuf, sem)
            cb = pltpu.async_copy(h3_hbm.at[ib.at[j]], bbuf, sem)
            ca.wait()
            cb.wait()

            def prod(i, _):
                r = i // 8
                kofs = (i % 8) * 16
                abuf[r, pl.ds(kofs, 16)] = (
                    abuf[r, pl.ds(kofs, 16)] * bbuf[r, pl.ds(kofs, 16)]
                )
                return _
            lax.fori_loop(0, 1024, prod, None)
            pltpu.sync_copy(abuf, out_hbm.at[pl.ds(base + j * 128, 128)])
            return _
        lax.fori_loop(0, LB, body, None)

    return k(la_r, lb_r, h3)


def _dinv_blk(d0_ref, d1_ref):
    deg = d0_ref[:, :1] + d1_ref[:, :1] + 1.0
    return lax.rsqrt(deg)


def _tc_first(x_p, W1, deg0, deg1):
    def body(x_ref, w_ref, d0_ref, d1_ref, o_ref):
        dinv = _dinv_blk(d0_ref, d1_ref)
        o_ref[...] = dinv * jnp.dot(x_ref[...], w_ref[...],
                                    preferred_element_type=jnp.float32)

    grid = NP // ROW_BLK
    return pl.pallas_call(
        body,
        grid=(grid,),
        in_specs=[
            pl.BlockSpec((ROW_BLK, D), lambda i: (i, 0)),
            pl.BlockSpec((D, D), lambda i: (0, 0)),
            pl.BlockSpec((ROW_BLK, 16), lambda i: (i, 0)),
            pl.BlockSpec((ROW_BLK, 16), lambda i: (i, 0)),
        ],
        out_specs=pl.BlockSpec((ROW_BLK, D), lambda i: (i, 0)),
        out_shape=jax.ShapeDtypeStruct((NP, D), jnp.float32),
    )(x_p, W1, deg0, deg1)


def _tc_mid(S0, S1, t, b, Wn, deg0, deg1):
    def body(s0_ref, s1_ref, t_ref, b_ref, w_ref, d0_ref, d1_ref, o_ref):
        dinv = _dinv_blk(d0_ref, d1_ref)
        u = dinv * (s0_ref[...] + s1_ref[...] + t_ref[...]) + b_ref[...]
        u = jnp.maximum(u, 0.0)
        o_ref[...] = dinv * jnp.dot(u, w_ref[...],
                                    preferred_element_type=jnp.float32)

    grid = NP // ROW_BLK
    return pl.pallas_call(
        body,
        grid=(grid,),
        in_specs=[
            pl.BlockSpec((ROW_BLK, D), lambda i: (i, 0)),
            pl.BlockSpec((ROW_BLK, D), lambda i: (i, 0)),
            pl.BlockSpec((ROW_BLK, D), lambda i: (i, 0)),
            pl.BlockSpec((1, D), lambda i: (0, 0)),
            pl.BlockSpec((D, D), lambda i: (0, 0)),
            pl.BlockSpec((ROW_BLK, 16), lambda i: (i, 0)),
            pl.BlockSpec((ROW_BLK, 16), lambda i: (i, 0)),
        ],
        out_specs=pl.BlockSpec((ROW_BLK, D), lambda i: (i, 0)),
        out_shape=jax.ShapeDtypeStruct((NP, D), jnp.float32),
    )(S0, S1, t, b, Wn, deg0, deg1)


def _tc_last(S0, S1, t, b, deg0, deg1):
    def body(s0_ref, s1_ref, t_ref, b_ref, d0_ref, d1_ref, o_ref):
        dinv = _dinv_blk(d0_ref, d1_ref)
        o_ref[...] = dinv * (s0_ref[...] + s1_ref[...] + t_ref[...]) + b_ref[...]

    grid = NP // ROW_BLK
    return pl.pallas_call(
        body,
        grid=(grid,),
        in_specs=[
            pl.BlockSpec((ROW_BLK, D), lambda i: (i, 0)),
            pl.BlockSpec((ROW_BLK, D), lambda i: (i, 0)),
            pl.BlockSpec((ROW_BLK, D), lambda i: (i, 0)),
            pl.BlockSpec((1, D), lambda i: (0, 0)),
            pl.BlockSpec((ROW_BLK, 16), lambda i: (i, 0)),
            pl.BlockSpec((ROW_BLK, 16), lambda i: (i, 0)),
        ],
        out_specs=pl.BlockSpec((ROW_BLK, D), lambda i: (i, 0)),
        out_shape=jax.ShapeDtypeStruct((NP, D), jnp.float32),
    )(S0, S1, t, b, deg0, deg1)


def _tc_rowsum(P):
    BLK = 2048

    def body(p_ref, o_ref):
        o_ref[...] = jnp.sum(p_ref[...], axis=1, keepdims=True)

    return pl.pallas_call(
        body,
        grid=(LP // BLK,),
        in_specs=[pl.BlockSpec((BLK, D), lambda i: (i, 0))],
        out_specs=pl.BlockSpec((BLK, 1), lambda i: (i, 0)),
        out_shape=jax.ShapeDtypeStruct((LP, 1), jnp.float32),
    )(P)


def kernel(x, edge_index, edge_label_index, W1, b1, W2, b2, W3, b3):
    x_p = jnp.pad(x, ((0, NP - N), (0, 0)))
    src_r = jnp.pad(edge_index[0], (0, EP - E)).reshape(NW, EB, 128)
    dst_r = jnp.pad(edge_index[1], (0, EP - E),
                    constant_values=DUMMY).reshape(NW, EB, 128)
    la_r = jnp.pad(edge_label_index[0], (0, LP - LBL)).reshape(NW, LB, 128)
    lb_r = jnp.pad(edge_label_index[1], (0, LP - LBL)).reshape(NW, LB, 128)
    b1r = b1.reshape(1, D)
    b2r = b2.reshape(1, D)
    b3r = b3.reshape(1, D)

    degs = _sc_degree(dst_r)
    deg0, deg1 = degs[0, :, :16], degs[1, :, :16]

    t1 = _tc_first(x_p, W1, deg0, deg1)
    S = _sc_propagate(src_r, dst_r, t1)
    t2 = _tc_mid(S[0], S[1], t1, b1r, W2, deg0, deg1)
    S = _sc_propagate(src_r, dst_r, t2)
    t3 = _tc_mid(S[0], S[1], t2, b2r, W3, deg0, deg1)
    S = _sc_propagate(src_r, dst_r, t3)
    h3 = _tc_last(S[0], S[1], t3, b3r, deg0, deg1)

    P = _sc_gather_prod(la_r, lb_r, h3)
    pred = _tc_rowsum(P)
    return pred.reshape(LP)[:LBL]


# speedup vs baseline: 5.7537x; 1.0439x over previous
import functools

import jax
import jax.numpy as jnp
from jax import lax
from jax.experimental import pallas as pl
from jax.experimental.pallas import tpu as pltpu
from jax.experimental.pallas import tpu_sc as plsc

N = 10000
NP = 10240
D = 128
NC = 2
NS = 16
NW = NC * NS
RPT = NP // NS

E = 320000
EB = 80
EP = NW * EB * 128
DUMMY = NP - 1

LBL = 100000
LB = 25
LP = NW * LB * 128

ROW_BLK = 1024


def _mesh():
    return plsc.VectorSubcoreMesh(
        core_axis_name="c", subcore_axis_name="s", num_cores=NC, num_subcores=NS
    )


def _fill_buf(buf, nrows, val):
    def body(i, _):
        r = i // 8
        k = (i % 8) * 16
        buf[r, pl.ds(k, 16)] = jnp.full((16,), val, jnp.float32)
        return _
    lax.fori_loop(0, nrows * 8, body, None)


def _sc_degree(dst_r):
    @functools.partial(
        pl.kernel,
        mesh=_mesh(),
        name="sc_degree",
        out_type=jax.ShapeDtypeStruct((NC, NP, D), jnp.float32),
        scratch_types=[
            pltpu.VMEM((EB, 128), jnp.int32),
            pltpu.VMEM((128, D), jnp.float32),
            pltpu.VMEM_SHARED((NP, D), jnp.float32),
            pltpu.SemaphoreType.DMA,
        ],
    )
    def k(dst_hbm, out_hbm, idx_v, ones_v, acc, ssem):
        c = lax.axis_index("c")
        s = lax.axis_index("s")
        w = c * NS + s

        _fill_buf(ones_v, 128, 0.0)
        for kk in range(RPT // 128):
            pltpu.sync_copy(ones_v, acc.at[pl.ds(s * RPT + kk * 128, 128)])
        _fill_buf(ones_v, 128, 1.0)
        pltpu.sync_copy(dst_hbm.at[w], idx_v)
        plsc.subcore_barrier()

        def body(g, _):
            for b in range(4):
                pltpu.async_copy(ones_v, acc.at[idx_v.at[g * 4 + b]], ssem,
                                 add=True)
            for b in range(4):
                pltpu.make_async_copy(ones_v, acc.at[idx_v.at[0]], ssem).wait()
            return _
        lax.fori_loop(0, EB // 4, body, None)

        plsc.subcore_barrier()
        pltpu.sync_copy(acc.at[pl.ds(s * RPT, RPT)],
                        out_hbm.at[c, pl.ds(s * RPT, RPT)])

    return k(dst_r)


def _sc_propagate(src_r, dst_r, t):
    @functools.partial(
        pl.kernel,
        mesh=_mesh(),
        name="sc_propagate",
        out_type=jax.ShapeDtypeStruct((NC, NP, D), jnp.float32),
        scratch_types=[
            pltpu.VMEM((EB, 128), jnp.int32),
            pltpu.VMEM((EB, 128), jnp.int32),
            pltpu.VMEM((128, D), jnp.float32),
            pltpu.VMEM_SHARED((NP, D), jnp.float32),
            pltpu.SemaphoreType.DMA,
        ],
    )
    def k(src_hbm, dst_hbm, t_hbm, out_hbm, idx_s, idx_d, rows, acc, gsem):
        c = lax.axis_index("c")
        s = lax.axis_index("s")
        w = c * NS + s

        _fill_buf(rows, 128, 0.0)
        for kk in range(RPT // 128):
            pltpu.sync_copy(rows, acc.at[pl.ds(s * RPT + kk * 128, 128)])
        pltpu.sync_copy(src_hbm.at[w], idx_s)
        pltpu.sync_copy(dst_hbm.at[w], idx_d)
        plsc.subcore_barrier()

        def body(j, _):
            pltpu.async_copy(t_hbm.at[idx_s.at[j]], rows, gsem).wait()
            pltpu.sync_copy(rows, acc.at[idx_d.at[j]], add=True)
            return _
        lax.fori_loop(0, EB, body, None)

        plsc.subcore_barrier()
        pltpu.sync_copy(acc.at[pl.ds(s * RPT, RPT)],
                        out_hbm.at[c, pl.ds(s * RPT, RPT)])

    return k(src_r, dst_r, t)


def _sc_gather_prod(la_r, lb_r, h3):
    @functools.partial(
        pl.kernel,
        mesh=_mesh(),
        name="sc_gather_prod",
        out_type=jax.ShapeDtypeStruct((LP, D), jnp.float32),
        scratch_types=[
            pltpu.VMEM((LB, 128), jnp.int32),
            pltpu.VMEM((LB, 128), jnp.int32),
            pltpu.VMEM((128, D), jnp.float32),
            pltpu.VMEM((128, D), jnp.float32),
            pltpu.VMEM((128, D), jnp.float32),
            pltpu.VMEM((128, D), jnp.float32),
            pltpu.SemaphoreType.DMA,
            pltpu.SemaphoreType.DMA,
        ],
    )
    def k(la_hbm, lb_hbm, h3_hbm, out_hbm, ia, ib, a0, a1, b0, b1,
          gsem, osem):
        abuf = [a0, a1]
        bbuf = [b0, b1]
        c = lax.axis_index("c")
        s = lax.axis_index("s")
        w = c * NS + s
        base = w * LB * 128

        pltpu.sync_copy(la_hbm.at[w], ia)
        pltpu.sync_copy(lb_hbm.at[w], ib)

        pltpu.async_copy(h3_hbm.at[ia.at[0]], abuf[0], gsem)
        pltpu.async_copy(h3_hbm.at[ib.at[0]], bbuf[0], gsem)
        for j in range(LB):
            cur = j % 2
            nxt = 1 - cur
            pltpu.make_async_copy(h3_hbm.at[ia.at[j]], abuf[cur], gsem).wait()
            pltpu.make_async_copy(h3_hbm.at[ib.at[j]], bbuf[cur], gsem).wait()
            if j + 1 < LB:
                pltpu.async_copy(h3_hbm.at[ia.at[j + 1]], abuf[nxt], gsem)
                pltpu.async_copy(h3_hbm.at[ib.at[j + 1]], bbuf[nxt], gsem)
            if j >= 2:
                pltpu.make_async_copy(abuf[cur],
                                      out_hbm.at[pl.ds(base, 128)],
                                      osem).wait()

            a, bb = abuf[cur], bbuf[cur]

            def prod(i, _):
                r = i // 2
                kofs = (i % 2) * 64
                for q in range(4):
                    o = kofs + q * 16
                    a[r, pl.ds(o, 16)] = a[r, pl.ds(o, 16)] * bb[r, pl.ds(o, 16)]
                return _
            lax.fori_loop(0, 256, prod, None)
            pltpu.async_copy(a, out_hbm.at[pl.ds(base + j * 128, 128)], osem)
        pltpu.make_async_copy(abuf[1], out_hbm.at[pl.ds(base, 128)],
                              osem).wait()
        pltpu.make_async_copy(abuf[0], out_hbm.at[pl.ds(base, 128)],
                              osem).wait()

    return k(la_r, lb_r, h3)


def _dinv_blk(d0_ref, d1_ref):
    deg = d0_ref[:, :1] + d1_ref[:, :1] + 1.0
    return lax.rsqrt(deg)


def _tc_first(x_p, W1, deg0, deg1):
    def body(x_ref, w_ref, d0_ref, d1_ref, o_ref):
        dinv = _dinv_blk(d0_ref, d1_ref)
        o_ref[...] = dinv * jnp.dot(x_ref[...], w_ref[...],
                                    preferred_element_type=jnp.float32)

    grid = NP // ROW_BLK
    return pl.pallas_call(
        body,
        grid=(grid,),
        in_specs=[
            pl.BlockSpec((ROW_BLK, D), lambda i: (i, 0)),
            pl.BlockSpec((D, D), lambda i: (0, 0)),
            pl.BlockSpec((ROW_BLK, 16), lambda i: (i, 0)),
            pl.BlockSpec((ROW_BLK, 16), lambda i: (i, 0)),
        ],
        out_specs=pl.BlockSpec((ROW_BLK, D), lambda i: (i, 0)),
        out_shape=jax.ShapeDtypeStruct((NP, D), jnp.float32),
    )(x_p, W1, deg0, deg1)


def _tc_mid(S0, S1, t, b, Wn, deg0, deg1):
    def body(s0_ref, s1_ref, t_ref, b_ref, w_ref, d0_ref, d1_ref, o_ref):
        dinv = _dinv_blk(d0_ref, d1_ref)
        u = dinv * (s0_ref[...] + s1_ref[...] + t_ref[...]) + b_ref[...]
        u = jnp.maximum(u, 0.0)
        o_ref[...] = dinv * jnp.dot(u, w_ref[...],
                                    preferred_element_type=jnp.float32)

    grid = NP // ROW_BLK
    return pl.pallas_call(
        body,
        grid=(grid,),
        in_specs=[
            pl.BlockSpec((ROW_BLK, D), lambda i: (i, 0)),
            pl.BlockSpec((ROW_BLK, D), lambda i: (i, 0)),
            pl.BlockSpec((ROW_BLK, D), lambda i: (i, 0)),
            pl.BlockSpec((1, D), lambda i: (0, 0)),
            pl.BlockSpec((D, D), lambda i: (0, 0)),
            pl.BlockSpec((ROW_BLK, 16), lambda i: (i, 0)),
            pl.BlockSpec((ROW_BLK, 16), lambda i: (i, 0)),
        ],
        out_specs=pl.BlockSpec((ROW_BLK, D), lambda i: (i, 0)),
        out_shape=jax.ShapeDtypeStruct((NP, D), jnp.float32),
    )(S0, S1, t, b, Wn, deg0, deg1)


def _tc_last(S0, S1, t, b, deg0, deg1):
    def body(s0_ref, s1_ref, t_ref, b_ref, d0_ref, d1_ref, o_ref):
        dinv = _dinv_blk(d0_ref, d1_ref)
        o_ref[...] = dinv * (s0_ref[...] + s1_ref[...] + t_ref[...]) + b_ref[...]

    grid = NP // ROW_BLK
    return pl.pallas_call(
        body,
        grid=(grid,),
        in_specs=[
            pl.BlockSpec((ROW_BLK, D), lambda i: (i, 0)),
            pl.BlockSpec((ROW_BLK, D), lambda i: (i, 0)),
            pl.BlockSpec((ROW_BLK, D), lambda i: (i, 0)),
            pl.BlockSpec((1, D), lambda i: (0, 0)),
            pl.BlockSpec((ROW_BLK, 16), lambda i: (i, 0)),
            pl.BlockSpec((ROW_BLK, 16), lambda i: (i, 0)),
        ],
        out_specs=pl.BlockSpec((ROW_BLK, D), lambda i: (i, 0)),
        out_shape=jax.ShapeDtypeStruct((NP, D), jnp.float32),
    )(S0, S1, t, b, deg0, deg1)


def _tc_rowsum(P):
    BLK = 2048

    def body(p_ref, o_ref):
        o_ref[...] = jnp.sum(p_ref[...], axis=1, keepdims=True)

    return pl.pallas_call(
        body,
        grid=(LP // BLK,),
        in_specs=[pl.BlockSpec((BLK, D), lambda i: (i, 0))],
        out_specs=pl.BlockSpec((BLK, 1), lambda i: (i, 0)),
        out_shape=jax.ShapeDtypeStruct((LP, 1), jnp.float32),
    )(P)


def kernel(x, edge_index, edge_label_index, W1, b1, W2, b2, W3, b3):
    x_p = jnp.pad(x, ((0, NP - N), (0, 0)))
    src_r = jnp.pad(edge_index[0], (0, EP - E)).reshape(NW, EB, 128)
    dst_r = jnp.pad(edge_index[1], (0, EP - E),
                    constant_values=DUMMY).reshape(NW, EB, 128)
    la_r = jnp.pad(edge_label_index[0], (0, LP - LBL)).reshape(NW, LB, 128)
    lb_r = jnp.pad(edge_label_index[1], (0, LP - LBL)).reshape(NW, LB, 128)
    b1r = b1.reshape(1, D)
    b2r = b2.reshape(1, D)
    b3r = b3.reshape(1, D)

    degs = _sc_degree(dst_r)
    deg0, deg1 = degs[0, :, :16], degs[1, :, :16]

    t1 = _tc_first(x_p, W1, deg0, deg1)
    S = _sc_propagate(src_r, dst_r, t1)
    t2 = _tc_mid(S[0], S[1], t1, b1r, W2, deg0, deg1)
    S = _sc_propagate(src_r, dst_r, t2)
    t3 = _tc_mid(S[0], S[1], t2, b2r, W3, deg0, deg1)
    S = _sc_propagate(src_r, dst_r, t3)
    h3 = _tc_last(S[0], S[1], t3, b3r, deg0, deg1)

    P = _sc_gather_prod(la_r, lb_r, h3)
    pred = _tc_rowsum(P)
    return pred.reshape(LP)[:LBL]


# speedup vs baseline: 14.4800x; 2.5167x over previous
import functools

import jax
import jax.numpy as jnp
from jax import lax
from jax.experimental import pallas as pl
from jax.experimental.pallas import tpu as pltpu
from jax.experimental.pallas import tpu_sc as plsc

N = 10000
NP = 10240
D = 128
NC = 2
NS = 16
NW = NC * NS
RPT = NP // NS

E = 320000
EB = 80
EP = NW * EB * 128
DUMMY = NP - 1

LBL = 100000
LB = 25
LP = NW * LB * 128

ROW_BLK = 1024


def _mesh():
    return plsc.VectorSubcoreMesh(
        core_axis_name="c", subcore_axis_name="s", num_cores=NC, num_subcores=NS
    )


def _fill_buf(buf, nrows, val):
    def body(i, _):
        r = i // 8
        k = (i % 8) * 16
        buf[r, pl.ds(k, 16)] = jnp.full((16,), val, jnp.float32)
        return _
    lax.fori_loop(0, nrows * 8, body, None)


def _sc_degree(dst_r):
    @functools.partial(
        pl.kernel,
        mesh=_mesh(),
        name="sc_degree",
        out_type=jax.ShapeDtypeStruct((NC, NP, D), jnp.float32),
        scratch_types=[
            pltpu.VMEM((EB, 128), jnp.int32),
            pltpu.VMEM((128, D), jnp.float32),
            pltpu.VMEM_SHARED((NP, D), jnp.float32),
            pltpu.SemaphoreType.DMA,
        ],
    )
    def k(dst_hbm, out_hbm, idx_v, ones_v, acc, ssem):
        c = lax.axis_index("c")
        s = lax.axis_index("s")
        w = c * NS + s

        _fill_buf(ones_v, 128, 0.0)
        for kk in range(RPT // 128):
            pltpu.sync_copy(ones_v, acc.at[pl.ds(s * RPT + kk * 128, 128)])
        _fill_buf(ones_v, 128, 1.0)
        pltpu.sync_copy(dst_hbm.at[w], idx_v)
        plsc.subcore_barrier()

        def body(g, _):
            for b in range(4):
                pltpu.async_copy(ones_v, acc.at[idx_v.at[g * 4 + b]], ssem,
                                 add=True)
            for b in range(4):
                pltpu.make_async_copy(ones_v, acc.at[idx_v.at[0]], ssem).wait()
            return _
        lax.fori_loop(0, EB // 4, body, None)

        plsc.subcore_barrier()
        pltpu.sync_copy(acc.at[pl.ds(s * RPT, RPT)],
                        out_hbm.at[c, pl.ds(s * RPT, RPT)])

    return k(dst_r)


def _sc_propagate(src_r, dst_r, t):
    @functools.partial(
        pl.kernel,
        mesh=_mesh(),
        name="sc_propagate",
        out_type=jax.ShapeDtypeStruct((NC, NP, D), jnp.float32),
        scratch_types=[
            pltpu.VMEM((EB, 128), jnp.int32),
            pltpu.VMEM((EB, 128), jnp.int32),
            pltpu.VMEM((128, D), jnp.float32),
            pltpu.VMEM_SHARED((NP, D), jnp.float32),
            pltpu.SemaphoreType.DMA,
        ],
    )
    def k(src_hbm, dst_hbm, t_hbm, out_hbm, idx_s, idx_d, rows, acc, gsem):
        c = lax.axis_index("c")
        s = lax.axis_index("s")
        w = c * NS + s

        _fill_buf(rows, 128, 0.0)
        for kk in range(RPT // 128):
            pltpu.sync_copy(rows, acc.at[pl.ds(s * RPT + kk * 128, 128)])
        pltpu.sync_copy(src_hbm.at[w], idx_s)
        pltpu.sync_copy(dst_hbm.at[w], idx_d)
        plsc.subcore_barrier()

        def body(j, _):
            pltpu.async_copy(t_hbm.at[idx_s.at[j]], rows, gsem).wait()
            pltpu.sync_copy(rows, acc.at[idx_d.at[j]], add=True)
            return _
        lax.fori_loop(0, EB, body, None)

        plsc.subcore_barrier()
        pltpu.sync_copy(acc.at[pl.ds(s * RPT, RPT)],
                        out_hbm.at[c, pl.ds(s * RPT, RPT)])

    return k(src_r, dst_r, t)


def _sc_gather_prod(la_r, lb_r, h3):
    @functools.partial(
        pl.kernel,
        mesh=_mesh(),
        name="sc_gather_prod",
        out_type=jax.ShapeDtypeStruct((LP, D), jnp.float32),
        scratch_types=[
            pltpu.VMEM((LB, 128), jnp.int32),
            pltpu.VMEM((LB, 128), jnp.int32),
            pltpu.VMEM((128, D), jnp.float32),
            pltpu.VMEM((128, D), jnp.float32),
            pltpu.VMEM((128, D), jnp.float32),
            pltpu.VMEM((128, D), jnp.float32),
            pltpu.SemaphoreType.DMA,
            pltpu.SemaphoreType.DMA,
        ],
    )
    def k(la_hbm, lb_hbm, h3_hbm, out_hbm, ia, ib, a0, a1, b0, b1,
          gsem, osem):
        abuf = [a0, a1]
        bbuf = [b0, b1]
        c = lax.axis_index("c")
        s = lax.axis_index("s")
        w = c * NS + s
        base = w * LB * 128

        pltpu.sync_copy(la_hbm.at[w], ia)
        pltpu.sync_copy(lb_hbm.at[w], ib)

        pltpu.async_copy(h3_hbm.at[ia.at[0]], abuf[0], gsem)
        pltpu.async_copy(h3_hbm.at[ib.at[0]], bbuf[0], gsem)
        for j in range(LB):
            cur = j % 2
            nxt = 1 - cur
            pltpu.make_async_copy(h3_hbm.at[ia.at[j]], abuf[cur], gsem).wait()
            pltpu.make_async_copy(h3_hbm.at[ib.at[j]], bbuf[cur], gsem).wait()
            if j + 1 < LB:
                pltpu.async_copy(h3_hbm.at[ia.at[j + 1]], abuf[nxt], gsem)
                pltpu.async_copy(h3_hbm.at[ib.at[j + 1]], bbuf[nxt], gsem)
            if j >= 2:
                pltpu.make_async_copy(abuf[cur],
                                      out_hbm.at[pl.ds(base, 128)],
                                      osem).wait()

            a, bb = abuf[cur], bbuf[cur]

            def prod(i, _):
                r = i // 2
                kofs = (i % 2) * 64
                for q in range(4):
                    o = kofs + q * 16
                    a[r, pl.ds(o, 16)] = a[r, pl.ds(o, 16)] * bb[r, pl.ds(o, 16)]
                return _
            lax.fori_loop(0, 256, prod, None)
            pltpu.async_copy(a, out_hbm.at[pl.ds(base + j * 128, 128)], osem)
        pltpu.make_async_copy(abuf[1], out_hbm.at[pl.ds(base, 128)],
                              osem).wait()
        pltpu.make_async_copy(abuf[0], out_hbm.at[pl.ds(base, 128)],
                              osem).wait()

    return k(la_r, lb_r, h3)


def _dinv_blk(d0_ref, d1_ref):
    deg = d0_ref[:, :1] + d1_ref[:, :1] + 1.0
    return lax.rsqrt(deg)


def _tc_first(x_p, W1, deg0, deg1):
    def body(x_ref, w_ref, d0_ref, d1_ref, o_ref):
        dinv = _dinv_blk(d0_ref, d1_ref)
        o_ref[...] = dinv * jnp.dot(x_ref[...], w_ref[...],
                                    preferred_element_type=jnp.float32)

    grid = NP // ROW_BLK
    return pl.pallas_call(
        body,
        grid=(grid,),
        in_specs=[
            pl.BlockSpec((ROW_BLK, D), lambda i: (i, 0)),
            pl.BlockSpec((D, D), lambda i: (0, 0)),
            pl.BlockSpec((ROW_BLK, 16), lambda i: (i, 0)),
            pl.BlockSpec((ROW_BLK, 16), lambda i: (i, 0)),
        ],
        out_specs=pl.BlockSpec((ROW_BLK, D), lambda i: (i, 0)),
        out_shape=jax.ShapeDtypeStruct((NP, D), jnp.float32),
    )(x_p, W1, deg0, deg1)


def _tc_mid(S0, S1, t, b, Wn, deg0, deg1):
    def body(s0_ref, s1_ref, t_ref, b_ref, w_ref, d0_ref, d1_ref, o_ref):
        dinv = _dinv_blk(d0_ref, d1_ref)
        u = dinv * (s0_ref[...] + s1_ref[...] + t_ref[...]) + b_ref[...]
        u = jnp.maximum(u, 0.0)
        o_ref[...] = dinv * jnp.dot(u, w_ref[...],
                                    preferred_element_type=jnp.float32)

    grid = NP // ROW_BLK
    return pl.pallas_call(
        body,
        grid=(grid,),
        in_specs=[
            pl.BlockSpec((ROW_BLK, D), lambda i: (i, 0)),
            pl.BlockSpec((ROW_BLK, D), lambda i: (i, 0)),
            pl.BlockSpec((ROW_BLK, D), lambda i: (i, 0)),
            pl.BlockSpec((1, D), lambda i: (0, 0)),
            pl.BlockSpec((D, D), lambda i: (0, 0)),
            pl.BlockSpec((ROW_BLK, 16), lambda i: (i, 0)),
            pl.BlockSpec((ROW_BLK, 16), lambda i: (i, 0)),
        ],
        out_specs=pl.BlockSpec((ROW_BLK, D), lambda i: (i, 0)),
        out_shape=jax.ShapeDtypeStruct((NP, D), jnp.float32),
    )(S0, S1, t, b, Wn, deg0, deg1)


def _tc_last(S0, S1, t, b, deg0, deg1):
    def body(s0_ref, s1_ref, t_ref, b_ref, d0_ref, d1_ref, o_ref):
        dinv = _dinv_blk(d0_ref, d1_ref)
        o_ref[...] = dinv * (s0_ref[...] + s1_ref[...] + t_ref[...]) + b_ref[...]

    grid = NP // ROW_BLK
    return pl.pallas_call(
        body,
        grid=(grid,),
        in_specs=[
            pl.BlockSpec((ROW_BLK, D), lambda i: (i, 0)),
            pl.BlockSpec((ROW_BLK, D), lambda i: (i, 0)),
            pl.BlockSpec((ROW_BLK, D), lambda i: (i, 0)),
            pl.BlockSpec((1, D), lambda i: (0, 0)),
            pl.BlockSpec((ROW_BLK, 16), lambda i: (i, 0)),
            pl.BlockSpec((ROW_BLK, 16), lambda i: (i, 0)),
        ],
        out_specs=pl.BlockSpec((ROW_BLK, D), lambda i: (i, 0)),
        out_shape=jax.ShapeDtypeStruct((NP, D), jnp.float32),
    )(S0, S1, t, b, deg0, deg1)


def _tc_rowsum(P):
    BLK = 2048

    def body(p_ref, o_ref):
        o_ref[...] = jnp.sum(p_ref[...], axis=1, keepdims=True)

    return pl.pallas_call(
        body,
        grid=(LP // BLK,),
        in_specs=[pl.BlockSpec((BLK, D), lambda i: (i, 0))],
        out_specs=pl.BlockSpec((BLK, 1), lambda i: (i, 0)),
        out_shape=jax.ShapeDtypeStruct((LP, 1), jnp.float32),
    )(P)


def kernel(x, edge_index, edge_label_index, W1, b1, W2, b2, W3, b3):
    x_p = jnp.pad(x, ((0, NP - N), (0, 0)))
    pad_i = jnp.arange(EP - E, dtype=jnp.int32)
    pad_src = pad_i % N
    pad_dst = N + (pad_i % (NP - N))
    src_r = jnp.concatenate([edge_index[0], pad_src]).reshape(NW, EB, 128)
    dst_r = jnp.concatenate([edge_index[1], pad_dst]).reshape(NW, EB, 128)
    pad_l = jnp.arange(LP - LBL, dtype=jnp.int32) % N
    la_r = jnp.concatenate([edge_label_index[0], pad_l]).reshape(NW, LB, 128)
    lb_r = jnp.concatenate([edge_label_index[1], pad_l]).reshape(NW, LB, 128)
    b1r = b1.reshape(1, D)
    b2r = b2.reshape(1, D)
    b3r = b3.reshape(1, D)

    degs = _sc_degree(dst_r)
    deg0, deg1 = degs[0, :, :16], degs[1, :, :16]

    t1 = _tc_first(x_p, W1, deg0, deg1)
    S = _sc_propagate(src_r, dst_r, t1)
    t2 = _tc_mid(S[0], S[1], t1, b1r, W2, deg0, deg1)
    S = _sc_propagate(src_r, dst_r, t2)
    t3 = _tc_mid(S[0], S[1], t2, b2r, W3, deg0, deg1)
    S = _sc_propagate(src_r, dst_r, t3)
    h3 = _tc_last(S[0], S[1], t3, b3r, deg0, deg1)

    P = _sc_gather_prod(la_r, lb_r, h3)
    pred = _tc_rowsum(P)
    return pred.reshape(LP)[:LBL]


# speedup vs baseline: 14.7772x; 1.0205x over previous
import functools

import numpy as np
import jax
import jax.numpy as jnp
from jax import lax
from jax.experimental import pallas as pl
from jax.experimental.pallas import tpu as pltpu
from jax.experimental.pallas import tpu_sc as plsc

N = 10000
NP = 10240
D = 128
NC = 2
NS = 16
NW = NC * NS
RPT = NP // NS

E = 320000
EB = 80
EP = NW * EB * 128
DUMMY = NP - 1

LBL = 100000
LB = 25
LP = NW * LB * 128

ROW_BLK = 1024


def _mesh():
    return plsc.VectorSubcoreMesh(
        core_axis_name="c", subcore_axis_name="s", num_cores=NC, num_subcores=NS
    )


def _fill_buf(buf, nrows, val):
    def body(i, _):
        r = i // 8
        k = (i % 8) * 16
        buf[r, pl.ds(k, 16)] = jnp.full((16,), val, jnp.float32)
        return _
    lax.fori_loop(0, nrows * 8, body, None)


def _sc_degree(dst_r):
    @functools.partial(
        pl.kernel,
        mesh=_mesh(),
        name="sc_degree",
        out_type=jax.ShapeDtypeStruct((NC, NP, D), jnp.float32),
        scratch_types=[
            pltpu.VMEM((EB, 128), jnp.int32),
            pltpu.VMEM((128, D), jnp.float32),
            pltpu.VMEM_SHARED((NP, D), jnp.float32),
            pltpu.SemaphoreType.DMA,
        ],
    )
    def k(dst_hbm, out_hbm, idx_v, ones_v, acc, ssem):
        c = lax.axis_index("c")
        s = lax.axis_index("s")
        w = c * NS + s

        _fill_buf(ones_v, 128, 0.0)
        for kk in range(RPT // 128):
            pltpu.sync_copy(ones_v, acc.at[pl.ds(s * RPT + kk * 128, 128)])
        _fill_buf(ones_v, 128, 1.0)
        pltpu.sync_copy(dst_hbm.at[w], idx_v)
        plsc.subcore_barrier()

        def body(g, _):
            for b in range(4):
                pltpu.async_copy(ones_v, acc.at[idx_v.at[g * 4 + b]], ssem,
                                 add=True)
            for b in range(4):
                pltpu.make_async_copy(ones_v, acc.at[idx_v.at[0]], ssem).wait()
            return _
        lax.fori_loop(0, EB // 4, body, None)

        plsc.subcore_barrier()
        pltpu.sync_copy(acc.at[pl.ds(s * RPT, RPT)],
                        out_hbm.at[c, pl.ds(s * RPT, RPT)])

    return k(dst_r)


def _sc_propagate(src_r, dst_r, t):
    @functools.partial(
        pl.kernel,
        mesh=_mesh(),
        name="sc_propagate",
        out_type=jax.ShapeDtypeStruct((NC, NP, D), jnp.float32),
        scratch_types=[
            pltpu.VMEM((EB, 128), jnp.int32),
            pltpu.VMEM((EB, 128), jnp.int32),
            pltpu.VMEM((128, D), jnp.float32),
            pltpu.VMEM_SHARED((NP, D), jnp.float32),
            pltpu.SemaphoreType.DMA,
        ],
    )
    def k(src_hbm, dst_hbm, t_hbm, out_hbm, idx_s, idx_d, rows, acc, gsem):
        c = lax.axis_index("c")
        s = lax.axis_index("s")
        w = c * NS + s

        _fill_buf(rows, 128, 0.0)
        for kk in range(RPT // 128):
            pltpu.sync_copy(rows, acc.at[pl.ds(s * RPT + kk * 128, 128)])
        pltpu.sync_copy(src_hbm.at[w], idx_s)
        pltpu.sync_copy(dst_hbm.at[w], idx_d)
        plsc.subcore_barrier()

        def body(j, _):
            pltpu.async_copy(t_hbm.at[idx_s.at[j]], rows, gsem).wait()
            pltpu.sync_copy(rows, acc.at[idx_d.at[j]], add=True)
            return _
        lax.fori_loop(0, EB, body, None)

        plsc.subcore_barrier()
        pltpu.sync_copy(acc.at[pl.ds(s * RPT, RPT)],
                        out_hbm.at[c, pl.ds(s * RPT, RPT)])

    return k(src_r, dst_r, t)


def _sc_gather_prod(la_r, lb_r, h3):
    @functools.partial(
        pl.kernel,
        mesh=_mesh(),
        name="sc_gather_prod",
        out_type=jax.ShapeDtypeStruct((LP, 16), jnp.float32),
        scratch_types=[
            pltpu.VMEM((LB, 128), jnp.int32),
            pltpu.VMEM((LB, 128), jnp.int32),
            pltpu.VMEM((128, D), jnp.float32),
            pltpu.VMEM((128, D), jnp.float32),
            pltpu.VMEM((128, D), jnp.float32),
            pltpu.VMEM((128, D), jnp.float32),
            pltpu.VMEM((128, 16), jnp.float32),
            pltpu.VMEM((128, 16), jnp.float32),
            pltpu.SemaphoreType.DMA,
            pltpu.SemaphoreType.DMA,
        ],
    )
    def k(la_hbm, lb_hbm, h3_hbm, out_hbm, ia, ib, a0, a1, b0, b1,
          p0, p1, gsem, osem):
        abuf = [a0, a1]
        bbuf = [b0, b1]
        pbuf = [p0, p1]
        c = lax.axis_index("c")
        s = lax.axis_index("s")
        w = c * NS + s
        base = w * LB * 128

        pltpu.sync_copy(la_hbm.at[w], ia)
        pltpu.sync_copy(lb_hbm.at[w], ib)

        pltpu.async_copy(h3_hbm.at[ia.at[0]], abuf[0], gsem)
        pltpu.async_copy(h3_hbm.at[ib.at[0]], bbuf[0], gsem)
        for j in range(LB):
            cur = j % 2
            nxt = 1 - cur
            pltpu.make_async_copy(h3_hbm.at[ia.at[j]], abuf[cur], gsem).wait()
            pltpu.make_async_copy(h3_hbm.at[ib.at[j]], bbuf[cur], gsem).wait()
            if j + 1 < LB:
                pltpu.async_copy(h3_hbm.at[ia.at[j + 1]], abuf[nxt], gsem)
                pltpu.async_copy(h3_hbm.at[ib.at[j + 1]], bbuf[nxt], gsem)
            if j >= 2:
                pltpu.make_async_copy(pbuf[cur],
                                      out_hbm.at[pl.ds(base, 128)],
                                      osem).wait()

            a, bb, pb = abuf[cur], bbuf[cur], pbuf[cur]

            def prod(p, _):
                acc = a[p, pl.ds(0, 16)] * bb[p, pl.ds(0, 16)]
                for q in range(1, 8):
                    o = q * 16
                    acc = acc + a[p, pl.ds(o, 16)] * bb[p, pl.ds(o, 16)]
                pb[p, :] = acc
                return _
            lax.fori_loop(0, 128, prod, None)
            pltpu.async_copy(pb, out_hbm.at[pl.ds(base + j * 128, 128)], osem)
        pltpu.make_async_copy(pbuf[1], out_hbm.at[pl.ds(base, 128)],
                              osem).wait()
        pltpu.make_async_copy(pbuf[0], out_hbm.at[pl.ds(base, 128)],
                              osem).wait()

    return k(la_r, lb_r, h3)


def _dinv_blk(d0_ref, d1_ref):
    deg = d0_ref[:, :1] + d1_ref[:, :1] + 1.0
    return lax.rsqrt(deg)


def _tc_first(x_p, W1, deg0, deg1):
    def body(x_ref, w_ref, d0_ref, d1_ref, o_ref):
        dinv = _dinv_blk(d0_ref, d1_ref)
        o_ref[...] = dinv * jnp.dot(x_ref[...], w_ref[...],
                                    preferred_element_type=jnp.float32)

    grid = NP // ROW_BLK
    return pl.pallas_call(
        body,
        grid=(grid,),
        in_specs=[
            pl.BlockSpec((ROW_BLK, D), lambda i: (i, 0)),
            pl.BlockSpec((D, D), lambda i: (0, 0)),
            pl.BlockSpec((ROW_BLK, 16), lambda i: (i, 0)),
            pl.BlockSpec((ROW_BLK, 16), lambda i: (i, 0)),
        ],
        out_specs=pl.BlockSpec((ROW_BLK, D), lambda i: (i, 0)),
        out_shape=jax.ShapeDtypeStruct((NP, D), jnp.float32),
    )(x_p, W1, deg0, deg1)


def _tc_mid(S0, S1, t, b, Wn, deg0, deg1):
    def body(s0_ref, s1_ref, t_ref, b_ref, w_ref, d0_ref, d1_ref, o_ref):
        dinv = _dinv_blk(d0_ref, d1_ref)
        u = dinv * (s0_ref[...] + s1_ref[...] + t_ref[...]) + b_ref[...]
        u = jnp.maximum(u, 0.0)
        o_ref[...] = dinv * jnp.dot(u, w_ref[...],
                                    preferred_element_type=jnp.float32)

    grid = NP // ROW_BLK
    return pl.pallas_call(
        body,
        grid=(grid,),
        in_specs=[
            pl.BlockSpec((ROW_BLK, D), lambda i: (i, 0)),
            pl.BlockSpec((ROW_BLK, D), lambda i: (i, 0)),
            pl.BlockSpec((ROW_BLK, D), lambda i: (i, 0)),
            pl.BlockSpec((1, D), lambda i: (0, 0)),
            pl.BlockSpec((D, D), lambda i: (0, 0)),
            pl.BlockSpec((ROW_BLK, 16), lambda i: (i, 0)),
            pl.BlockSpec((ROW_BLK, 16), lambda i: (i, 0)),
        ],
        out_specs=pl.BlockSpec((ROW_BLK, D), lambda i: (i, 0)),
        out_shape=jax.ShapeDtypeStruct((NP, D), jnp.float32),
    )(S0, S1, t, b, Wn, deg0, deg1)


def _tc_last(S0, S1, t, b, deg0, deg1):
    def body(s0_ref, s1_ref, t_ref, b_ref, d0_ref, d1_ref, o_ref):
        dinv = _dinv_blk(d0_ref, d1_ref)
        o_ref[...] = dinv * (s0_ref[...] + s1_ref[...] + t_ref[...]) + b_ref[...]

    grid = NP // ROW_BLK
    return pl.pallas_call(
        body,
        grid=(grid,),
        in_specs=[
            pl.BlockSpec((ROW_BLK, D), lambda i: (i, 0)),
            pl.BlockSpec((ROW_BLK, D), lambda i: (i, 0)),
            pl.BlockSpec((ROW_BLK, D), lambda i: (i, 0)),
            pl.BlockSpec((1, D), lambda i: (0, 0)),
            pl.BlockSpec((ROW_BLK, 16), lambda i: (i, 0)),
            pl.BlockSpec((ROW_BLK, 16), lambda i: (i, 0)),
        ],
        out_specs=pl.BlockSpec((ROW_BLK, D), lambda i: (i, 0)),
        out_shape=jax.ShapeDtypeStruct((NP, D), jnp.float32),
    )(S0, S1, t, b, deg0, deg1)


def _tc_rowsum(P):
    BLK = 4096

    def body(p_ref, o_ref):
        o_ref[...] = jnp.sum(p_ref[...], axis=1, keepdims=True)

    return pl.pallas_call(
        body,
        grid=(LP // BLK,),
        in_specs=[pl.BlockSpec((BLK, 16), lambda i: (i, 0))],
        out_specs=pl.BlockSpec((BLK, 1), lambda i: (i, 0)),
        out_shape=jax.ShapeDtypeStruct((LP, 1), jnp.float32),
    )(P)


def kernel(x, edge_index, edge_label_index, W1, b1, W2, b2, W3, b3):
    x_p = jnp.pad(x, ((0, NP - N), (0, 0)))
    pad_i = np.arange(EP - E, dtype=np.int32)
    pad_src = jnp.asarray(pad_i % N)
    pad_dst = jnp.asarray(N + (pad_i % (NP - N)))
    src_r = jnp.concatenate([edge_index[0], pad_src]).reshape(NW, EB, 128)
    dst_r = jnp.concatenate([edge_index[1], pad_dst]).reshape(NW, EB, 128)
    pad_l = jnp.asarray(np.arange(LP - LBL, dtype=np.int32) % N)
    la_r = jnp.concatenate([edge_label_index[0], pad_l]).reshape(NW, LB, 128)
    lb_r = jnp.concatenate([edge_label_index[1], pad_l]).reshape(NW, LB, 128)
    b1r = b1.reshape(1, D)
    b2r = b2.reshape(1, D)
    b3r = b3.reshape(1, D)

    degs = _sc_degree(dst_r)
    deg0, deg1 = degs[0, :, :16], degs[1, :, :16]

    t1 = _tc_first(x_p, W1, deg0, deg1)
    S = _sc_propagate(src_r, dst_r, t1)
    t2 = _tc_mid(S[0], S[1], t1, b1r, W2, deg0, deg1)
    S = _sc_propagate(src_r, dst_r, t2)
    t3 = _tc_mid(S[0], S[1], t2, b2r, W3, deg0, deg1)
    S = _sc_propagate(src_r, dst_r, t3)
    h3 = _tc_last(S[0], S[1], t3, b3r, deg0, deg1)

    P = _sc_gather_prod(la_r, lb_r, h3)
    pred = _tc_rowsum(P)
    return pred.reshape(LP)[:LBL]


# speedup vs baseline: 14.7838x; 1.0004x over previous
import functools

import numpy as np
import jax
import jax.numpy as jnp
from jax import lax
from jax.experimental import pallas as pl
from jax.experimental.pallas import tpu as pltpu
from jax.experimental.pallas import tpu_sc as plsc

N = 10000
NP = 10240
D = 128
NC = 2
NS = 16
NW = NC * NS
RPT = NP // NS

E = 320000
EB = 80
EP = NW * EB * 128
DUMMY = NP - 1

LBL = 100000
LB = 25
LP = NW * LB * 128

ROW_BLK = 1024


def _mesh():
    return plsc.VectorSubcoreMesh(
        core_axis_name="c", subcore_axis_name="s", num_cores=NC, num_subcores=NS
    )


def _fill_buf(buf, nrows, val):
    def body(i, _):
        r = i // 8
        k = (i % 8) * 16
        buf[r, pl.ds(k, 16)] = jnp.full((16,), val, jnp.float32)
        return _
    lax.fori_loop(0, nrows * 8, body, None)


def _sc_degree(dst_r):
    @functools.partial(
        pl.kernel,
        mesh=_mesh(),
        name="sc_degree",
        out_type=jax.ShapeDtypeStruct((NC, NP, D), jnp.float32),
        scratch_types=[
            pltpu.VMEM((EB, 128), jnp.int32),
            pltpu.VMEM((128, D), jnp.float32),
            pltpu.VMEM_SHARED((NP, D), jnp.float32),
            pltpu.SemaphoreType.DMA,
        ],
    )
    def k(dst_hbm, out_hbm, idx_v, ones_v, acc, ssem):
        c = lax.axis_index("c")
        s = lax.axis_index("s")
        w = c * NS + s

        _fill_buf(ones_v, 128, 0.0)
        for kk in range(RPT // 128):
            pltpu.sync_copy(ones_v, acc.at[pl.ds(s * RPT + kk * 128, 128)])
        _fill_buf(ones_v, 128, 1.0)
        pltpu.sync_copy(dst_hbm.at[w], idx_v)
        plsc.subcore_barrier()

        def body(g, _):
            for b in range(4):
                pltpu.async_copy(ones_v, acc.at[idx_v.at[g * 4 + b]], ssem,
                                 add=True)
            for b in range(4):
                pltpu.make_async_copy(ones_v, acc.at[idx_v.at[0]], ssem).wait()
            return _
        lax.fori_loop(0, EB // 4, body, None)

        plsc.subcore_barrier()
        pltpu.sync_copy(acc.at[pl.ds(s * RPT, RPT)],
                        out_hbm.at[c, pl.ds(s * RPT, RPT)])

    return k(dst_r)


def _sc_propagate(src_r, dst_r, t):
    @functools.partial(
        pl.kernel,
        mesh=_mesh(),
        name="sc_propagate",
        out_type=jax.ShapeDtypeStruct((NC, NP, D), jnp.float32),
        scratch_types=[
            pltpu.VMEM((EB, 128), jnp.int32),
            pltpu.VMEM((EB, 128), jnp.int32),
            pltpu.VMEM((128, D), jnp.float32),
            pltpu.VMEM_SHARED((NP, D), jnp.float32),
            pltpu.SemaphoreType.DMA,
        ],
    )
    def k(src_hbm, dst_hbm, t_hbm, out_hbm, idx_s, idx_d, rows, acc, gsem):
        c = lax.axis_index("c")
        s = lax.axis_index("s")
        w = c * NS + s

        _fill_buf(rows, 128, 0.0)
        for kk in range(RPT // 128):
            pltpu.sync_copy(rows, acc.at[pl.ds(s * RPT + kk * 128, 128)])
        pltpu.sync_copy(src_hbm.at[w], idx_s)
        pltpu.sync_copy(dst_hbm.at[w], idx_d)
        plsc.subcore_barrier()

        def body(j, _):
            pltpu.async_copy(t_hbm.at[idx_s.at[j]], rows, gsem).wait()
            pltpu.sync_copy(rows, acc.at[idx_d.at[j]], add=True)
            return _
        lax.fori_loop(0, EB, body, None)

        plsc.subcore_barrier()
        pltpu.sync_copy(acc.at[pl.ds(s * RPT, RPT)],
                        out_hbm.at[c, pl.ds(s * RPT, RPT)])

    return k(src_r, dst_r, t)


def _sc_gather_prod(la_r, lb_r, h3):
    @functools.partial(
        pl.kernel,
        mesh=_mesh(),
        name="sc_gather_prod",
        out_type=jax.ShapeDtypeStruct((LP, 16), jnp.float32),
        scratch_types=[
            pltpu.VMEM((LB, 128), jnp.int32),
            pltpu.VMEM((LB, 128), jnp.int32),
            pltpu.VMEM((128, D), jnp.float32),
            pltpu.VMEM((128, D), jnp.float32),
            pltpu.VMEM((128, D), jnp.float32),
            pltpu.VMEM((128, D), jnp.float32),
            pltpu.VMEM((128, 16), jnp.float32),
            pltpu.VMEM((128, 16), jnp.float32),
            pltpu.SemaphoreType.DMA,
            pltpu.SemaphoreType.DMA,
        ],
    )
    def k(la_hbm, lb_hbm, h3_hbm, out_hbm, ia, ib, a0, a1, b0, b1,
          p0, p1, gsem, osem):
        abuf = [a0, a1]
        bbuf = [b0, b1]
        pbuf = [p0, p1]
        c = lax.axis_index("c")
        s = lax.axis_index("s")
        w = c * NS + s
        base = w * LB * 128

        pltpu.sync_copy(la_hbm.at[w], ia)
        pltpu.sync_copy(lb_hbm.at[w], ib)

        pltpu.async_copy(h3_hbm.at[ia.at[0]], abuf[0], gsem)
        pltpu.async_copy(h3_hbm.at[ib.at[0]], bbuf[0], gsem)
        for j in range(LB):
            cur = j % 2
            nxt = 1 - cur
            pltpu.make_async_copy(h3_hbm.at[ia.at[j]], abuf[cur], gsem).wait()
            pltpu.make_async_copy(h3_hbm.at[ib.at[j]], bbuf[cur], gsem).wait()
            if j + 1 < LB:
                pltpu.async_copy(h3_hbm.at[ia.at[j + 1]], abuf[nxt], gsem)
                pltpu.async_copy(h3_hbm.at[ib.at[j + 1]], bbuf[nxt], gsem)
            if j >= 2:
                pltpu.make_async_copy(pbuf[cur],
                                      out_hbm.at[pl.ds(base, 128)],
                                      osem).wait()

            a, bb, pb = abuf[cur], bbuf[cur], pbuf[cur]

            def prod(p, _):
                acc = a[p, pl.ds(0, 16)] * bb[p, pl.ds(0, 16)]
                for q in range(1, 8):
                    o = q * 16
                    acc = acc + a[p, pl.ds(o, 16)] * bb[p, pl.ds(o, 16)]
                pb[p, :] = acc
                return _
            lax.fori_loop(0, 128, prod, None)
            pltpu.async_copy(pb, out_hbm.at[pl.ds(base + j * 128, 128)], osem)
        pltpu.make_async_copy(pbuf[1], out_hbm.at[pl.ds(base, 128)],
                              osem).wait()
        pltpu.make_async_copy(pbuf[0], out_hbm.at[pl.ds(base, 128)],
                              osem).wait()

    return k(la_r, lb_r, h3)


def _dinv_blk(d0_ref, d1_ref):
    deg = d0_ref[:, :1] + d1_ref[:, :1] + 1.0
    return lax.rsqrt(deg)


def _tc_first(x_p, W1, deg0, deg1):
    def body(x_ref, w_ref, d0_ref, d1_ref, o_ref):
        dinv = _dinv_blk(d0_ref, d1_ref)
        o_ref[...] = dinv * jnp.dot(x_ref[...], w_ref[...],
                                    preferred_element_type=jnp.float32)

    grid = NP // ROW_BLK
    return pl.pallas_call(
        body,
        grid=(grid,),
        in_specs=[
            pl.BlockSpec((ROW_BLK, D), lambda i: (i, 0)),
            pl.BlockSpec((D, D), lambda i: (0, 0)),
            pl.BlockSpec((ROW_BLK, 16), lambda i: (i, 0)),
            pl.BlockSpec((ROW_BLK, 16), lambda i: (i, 0)),
        ],
        out_specs=pl.BlockSpec((ROW_BLK, D), lambda i: (i, 0)),
        out_shape=jax.ShapeDtypeStruct((N, D), jnp.float32),
    )(x_p, W1, deg0, deg1)


def _tc_mid(S0, S1, t, b, Wn, deg0, deg1):
    def body(s0_ref, s1_ref, t_ref, b_ref, w_ref, d0_ref, d1_ref, o_ref):
        dinv = _dinv_blk(d0_ref, d1_ref)
        u = dinv * (s0_ref[...] + s1_ref[...] + t_ref[...]) + b_ref[...]
        u = jnp.maximum(u, 0.0)
        o_ref[...] = dinv * jnp.dot(u, w_ref[...],
                                    preferred_element_type=jnp.float32)

    grid = NP // ROW_BLK
    return pl.pallas_call(
        body,
        grid=(grid,),
        in_specs=[
            pl.BlockSpec((ROW_BLK, D), lambda i: (i, 0)),
            pl.BlockSpec((ROW_BLK, D), lambda i: (i, 0)),
            pl.BlockSpec((ROW_BLK, D), lambda i: (i, 0)),
            pl.BlockSpec((1, D), lambda i: (0, 0)),
            pl.BlockSpec((D, D), lambda i: (0, 0)),
            pl.BlockSpec((ROW_BLK, 16), lambda i: (i, 0)),
            pl.BlockSpec((ROW_BLK, 16), lambda i: (i, 0)),
        ],
        out_specs=pl.BlockSpec((ROW_BLK, D), lambda i: (i, 0)),
        out_shape=jax.ShapeDtypeStruct((N, D), jnp.float32),
    )(S0, S1, t, b, Wn, deg0, deg1)


def _tc_last(S0, S1, t, b, deg0, deg1):
    def body(s0_ref, s1_ref, t_ref, b_ref, d0_ref, d1_ref, o_ref):
        dinv = _dinv_blk(d0_ref, d1_ref)
        o_ref[...] = dinv * (s0_ref[...] + s1_ref[...] + t_ref[...]) + b_ref[...]

    grid = NP // ROW_BLK
    return pl.pallas_call(
        body,
        grid=(grid,),
        in_specs=[
            pl.BlockSpec((ROW_BLK, D), lambda i: (i, 0)),
            pl.BlockSpec((ROW_BLK, D), lambda i: (i, 0)),
            pl.BlockSpec((ROW_BLK, D), lambda i: (i, 0)),
            pl.BlockSpec((1, D), lambda i: (0, 0)),
            pl.BlockSpec((ROW_BLK, 16), lambda i: (i, 0)),
            pl.BlockSpec((ROW_BLK, 16), lambda i: (i, 0)),
        ],
        out_specs=pl.BlockSpec((ROW_BLK, D), lambda i: (i, 0)),
        out_shape=jax.ShapeDtypeStruct((N, D), jnp.float32),
    )(S0, S1, t, b, deg0, deg1)


def _tc_rowsum(P):
    BLK = 4096

    def body(p_ref, o_ref):
        o_ref[...] = jnp.sum(p_ref[...], axis=1, keepdims=True)

    return pl.pallas_call(
        body,
        grid=(LP // BLK,),
        in_specs=[pl.BlockSpec((BLK, 16), lambda i: (i, 0))],
        out_specs=pl.BlockSpec((BLK, 1), lambda i: (i, 0)),
        out_shape=jax.ShapeDtypeStruct((LP, 1), jnp.float32),
    )(P)


def kernel(x, edge_index, edge_label_index, W1, b1, W2, b2, W3, b3):
    x_p = x
    pad_i = np.arange(EP - E, dtype=np.int32)
    pad_src = jnp.asarray(pad_i % N)
    pad_dst = jnp.asarray(N + (pad_i % (NP - N)))
    src_r = jnp.concatenate([edge_index[0], pad_src]).reshape(NW, EB, 128)
    dst_r = jnp.concatenate([edge_index[1], pad_dst]).reshape(NW, EB, 128)
    pad_l = jnp.asarray(np.arange(LP - LBL, dtype=np.int32) % N)
    la_r = jnp.concatenate([edge_label_index[0], pad_l]).reshape(NW, LB, 128)
    lb_r = jnp.concatenate([edge_label_index[1], pad_l]).reshape(NW, LB, 128)
    b1r = b1.reshape(1, D)
    b2r = b2.reshape(1, D)
    b3r = b3.reshape(1, D)

    degs = _sc_degree(dst_r)
    deg0, deg1 = degs[0, :, :16], degs[1, :, :16]

    t1 = _tc_first(x_p, W1, deg0, deg1)
    S = _sc_propagate(src_r, dst_r, t1)
    t2 = _tc_mid(S[0], S[1], t1, b1r, W2, deg0, deg1)
    S = _sc_propagate(src_r, dst_r, t2)
    t3 = _tc_mid(S[0], S[1], t2, b2r, W3, deg0, deg1)
    S = _sc_propagate(src_r, dst_r, t3)
    h3 = _tc_last(S[0], S[1], t3, b3r, deg0, deg1)

    P = _sc_gather_prod(la_r, lb_r, h3)
    pred = _tc_rowsum(P)
    return pred.reshape(LP)[:LBL]
